# baseline (speedup 1.0000x reference)
import functools

import jax
import jax.numpy as jnp
from jax import lax
from jax.experimental import pallas as pl
from jax.experimental.pallas import tpu as pltpu
from jax.experimental.pallas import tpu_sc as plsc

RADIUS = 0.1
NSAMPLE = 16
EPS = 1e-5


def _k1_body(p_ref, ps_ref, ft_ref, w3_ref, wf_ref, g_ref, hq_ref):
    pb = p_ref[0]
    psb = ps_ref[0]
    ftb = ft_ref[0]
    gp = jnp.dot(pb, w3_ref[...], preferred_element_type=jnp.float32)
    g_ref[0] = gp + jnp.dot(ftb, wf_ref[...],
                            preferred_element_type=jnp.float32)
    hq_ref[0] = jnp.dot(psb, w3_ref[...], preferred_element_type=jnp.float32)


def _k1(p, ps, fT, w3, wf):
    B, N, C = fT.shape
    Q = 1024
    grid = (B, N // Q)
    return pl.pallas_call(
        _k1_body,
        grid=grid,
        in_specs=[
            pl.BlockSpec((1, Q, 3), lambda b, q: (b, q, 0)),
            pl.BlockSpec((1, Q, 3), lambda b, q: (b, q, 0)),
            pl.BlockSpec((1, Q, C), lambda b, q: (b, q, 0)),
            pl.BlockSpec((3, C), lambda b, q: (0, 0)),
            pl.BlockSpec((C, C), lambda b, q: (0, 0)),
        ],
        out_specs=[
            pl.BlockSpec((1, Q, C), lambda b, q: (b, q, 0)),
            pl.BlockSpec((1, Q, C), lambda b, q: (b, q, 0)),
        ],
        out_shape=[
            jax.ShapeDtypeStruct((B, N, C), jnp.float32),
            jax.ShapeDtypeStruct((B, N, C), jnp.float32),
        ],
    )(p, ps, fT, w3, wf)


def _k2(table, idx_flat):
    info = plsc.get_sparse_core_info()
    nw = info.num_cores * info.num_subcores
    R = idx_flat.shape[0]
    D = table.shape[1]
    per_w = R // nw
    CH = 256
    nch = per_w // CH
    nc = info.num_cores
    mesh = plsc.VectorSubcoreMesh(core_axis_name="c", subcore_axis_name="s")

    @functools.partial(
        pl.kernel, mesh=mesh,
        out_type=jax.ShapeDtypeStruct((R, D), jnp.float32),
        scratch_types=[
            pltpu.VMEM((CH,), jnp.int32),
            pltpu.VMEM((CH, D), jnp.float32),
            pltpu.SemaphoreType.DMA,
        ],
    )
    def k(table_hbm, idx_hbm, out_hbm, idx_v, rows_v, sem):
        wid = lax.axis_index("s") * nc + lax.axis_index("c")
        base0 = wid * per_w

        def body(c, carry):
            base = base0 + c * CH
            pltpu.sync_copy(idx_hbm.at[pl.ds(base, CH)], idx_v)
            pltpu.async_copy(table_hbm.at[idx_v], rows_v, sem).wait()
            pltpu.sync_copy(rows_v, out_hbm.at[pl.ds(base, CH)])
            return carry

        lax.fori_loop(0, nch, body, 0)

    return k(table, idx_flat)


def _k3_body(g_ref, hq_ref, ymax_ref, ysum_ref, ysq_ref):
    g = g_ref[0]
    hq = hq_ref[0]
    gmax = jnp.max(g, axis=1)
    gsum = jnp.sum(g, axis=1)
    gsq = jnp.sum(g * g, axis=1)
    ymax_ref[0] = gmax - hq
    k = jnp.float32(NSAMPLE)
    ysum_ref[0] = jnp.sum(gsum - k * hq, axis=0, keepdims=True)
    ysq_ref[0] = jnp.sum(gsq - 2.0 * hq * gsum + k * hq * hq,
                         axis=0, keepdims=True)


def _k3(gathered, Hq):
    B, N, K, C = gathered.shape
    Q = 512
    nq = N // Q
    grid = (B, nq)
    return pl.pallas_call(
        _k3_body,
        grid=grid,
        in_specs=[
            pl.BlockSpec((1, Q, K, C), lambda b, q: (b, q, 0, 0)),
            pl.BlockSpec((1, Q, C), lambda b, q: (b, q, 0)),
        ],
        out_specs=[
            pl.BlockSpec((1, Q, C), lambda b, q: (b, q, 0)),
            pl.BlockSpec((1, 1, C), lambda b, q, _nq=nq: (b * _nq + q, 0, 0)),
            pl.BlockSpec((1, 1, C), lambda b, q, _nq=nq: (b * _nq + q, 0, 0)),
        ],
        out_shape=[
            jax.ShapeDtypeStruct((B, N, C), jnp.float32),
            jax.ShapeDtypeStruct((B * nq, 1, C), jnp.float32),
            jax.ShapeDtypeStruct((B * nq, 1, C), jnp.float32),
        ],
    )(gathered, Hq)


def _k456_body(ymax_ref, ft_ref, ysum_ref, ysq_ref, w1t_ref, w2t_ref,
               g0_ref, b0_ref, g1_ref, b1_ref, g2_ref, b2_ref, out_ref,
               *, n0, n1):
    ymax = ymax_ref[...]
    ft = ft_ref[...]
    m0 = jnp.sum(ysum_ref[...], axis=0, keepdims=True) / n0
    v0 = jnp.sum(ysq_ref[...], axis=0, keepdims=True) / n0 - m0 * m0
    r0 = g0_ref[...] * lax.rsqrt(v0 + EPS)
    fa = jnp.maximum((ymax - m0) * r0 + b0_ref[...], 0.0)
    h1 = jnp.dot(fa, w1t_ref[...], preferred_element_type=jnp.float32)
    m1 = jnp.sum(h1, axis=0, keepdims=True) / n1
    v1 = jnp.sum(h1 * h1, axis=0, keepdims=True) / n1 - m1 * m1
    r1 = g1_ref[...] * lax.rsqrt(v1 + EPS)
    h1n = jnp.maximum((h1 - m1) * r1 + b1_ref[...], 0.0)
    h2 = jnp.dot(h1n, w2t_ref[...], preferred_element_type=jnp.float32)
    m2 = jnp.sum(h2, axis=0, keepdims=True) / n1
    v2 = jnp.sum(h2 * h2, axis=0, keepdims=True) / n1 - m2 * m2
    r2 = g2_ref[...] * lax.rsqrt(v2 + EPS)
    h2n = (h2 - m2) * r2 + b2_ref[...]
    out_ref[...] = jnp.maximum(h2n + ft, 0.0)


def _k456(ymax, ftf, ysum, ysq, w1t, w2t, g0, b0, g1, b1, g2, b2):
    M, C = ymax.shape
    n0 = float(M * NSAMPLE)
    n1 = float(M)
    row = lambda v: v.reshape(1, C)
    return pl.pallas_call(
        functools.partial(_k456_body, n0=n0, n1=n1),
        out_shape=jax.ShapeDtypeStruct((M, C), jnp.float32),
    )(ymax, ftf, ysum, ysq, w1t, w2t,
      row(g0), row(b0), row(g1), row(b1), row(g2), row(b2))


def _merge16(pos, blk):
    m = [jnp.minimum(pos[i], blk[15 - i]) for i in range(16)]
    for stride in (8, 4, 2, 1):
        nm = list(m)
        for i in range(16):
            if (i % (2 * stride)) < stride:
                lo = jnp.minimum(m[i], m[i + stride])
                hi = jnp.maximum(m[i], m[i + stride])
                nm[i] = lo
                nm[i + stride] = hi
        m = nm
    return m


def _k0s_body(lo_ref, hi_ref, pts_ref, ido_ref, idx_ref,
              *, Q, JB, N, W, nsample):
    b = pl.program_id(0)
    qt = pl.program_id(1)
    q = pts_ref[0, :, pl.ds(qt * Q, Q)]
    qx = q[0, :].reshape(Q, 1)
    qy = q[1, :].reshape(Q, 1)
    qz = q[2, :].reshape(Q, 1)
    r2 = jnp.float32(RADIUS * RADIUS)
    big = jnp.float32(N)

    def scan_block(start, pos):
        cols = pts_ref[0, :, pl.ds(start, JB)]
        oid = ido_ref[0, 0:1, pl.ds(start, JB)]
        cx = cols[0, :].reshape(1, JB)
        cy = cols[1, :].reshape(1, JB)
        cz = cols[2, :].reshape(1, JB)
        dx = qx - cx
        dy = qy - cy
        dz = qz - cz
        d2 = dx * dx + dy * dy + dz * dz
        ov = jnp.where(d2 <= r2, oid, big)
        prev = jnp.full((Q, 1), jnp.float32(-1.0))
        blk = []
        for _ in range(nsample):
            hit = jnp.where(ov > prev, ov, big)
            cand = jnp.min(hit, axis=1, keepdims=True)
            blk.append(cand)
            prev = cand
        return tuple(_merge16(list(pos), blk))

    init = tuple(jnp.full((Q, 1), big) for _ in range(nsample))
    t = b * (pl.num_programs(1)) + qt
    lo0 = jnp.minimum(lo_ref[t] * JB, N - W)
    pos = init
    for i in range(W // JB):
        pos = scan_block(lo0 + i * JB, pos)
    rem_lo = lo0 // JB + (W // JB)
    pos = lax.fori_loop(rem_lo, hi_ref[t],
                        lambda cb, pp: scan_block(cb * JB, pp), pos)
    first = pos[0]
    base = b * N
    for s in range(nsample):
        sel = jnp.where(pos[s] < big, pos[s], first)
        idx_ref[0, :, s:s + 1] = sel.astype(jnp.int32) + base


def _k0s(pTs, ido, cb_lo, cb_hi, nsample):
    B, _, N = pTs.shape
    Q = _K0_Q
    JB = _K0_JB
    grid = (B, N // Q)
    return pl.pallas_call(
        functools.partial(_k0s_body, Q=Q, JB=JB, N=N, W=_K0_W,
                          nsample=nsample),
        grid=grid,
        in_specs=[
            pl.BlockSpec(memory_space=pltpu.SMEM),
            pl.BlockSpec(memory_space=pltpu.SMEM),
            pl.BlockSpec((1, 3, N), lambda b, q: (b, 0, 0)),
            pl.BlockSpec((1, 1, N), lambda b, q: (b, 0, 0)),
        ],
        out_specs=pl.BlockSpec((1, Q, nsample), lambda b, q: (b, q, 0)),
        out_shape=jax.ShapeDtypeStruct((B, N, nsample), jnp.int32),
    )(cb_lo, cb_hi, pTs, ido)


def _ball_query_xla(p, radius, nsample):
    B, N, _ = p.shape
    r2 = radius * radius
    order = jnp.arange(N, dtype=jnp.float32)[None, None, :]
    chunk = 1024
    idx_chunks = []
    for s in range(0, N, chunk):
        q = p[:, s:s + chunk]
        d2 = jnp.sum((q[:, :, None, :] - p[:, None, :, :]) ** 2, axis=-1)
        valid = d2 <= r2
        score = jnp.where(valid, order, float(N))
        neg_vals, idx = lax.top_k(-score, nsample)
        found = (-neg_vals) < float(N)
        firstc = idx[:, :, :1]
        idx = jnp.where(found, idx, firstc)
        idx_chunks.append(idx)
    return jnp.concatenate(idx_chunks, axis=1)


_K0_Q = 1024
_K0_JB = 1024
_K0_W = 4096


def kernel(p, f, W0, g0, b0, W1, g1, b1, W2, g2, b2):
    B, C, N = f.shape
    fT = jnp.transpose(f, (0, 2, 1))
    w3 = jnp.transpose(W0[:, :3])
    wf = jnp.transpose(W0[:, 3:])

    boff = (jnp.arange(B, dtype=jnp.int32) * N)[:, None]
    perm = jnp.argsort(p[..., 0], axis=1).astype(jnp.int32)
    permg = (perm + boff).reshape(-1)
    inv = jnp.argsort(perm, axis=1).astype(jnp.int32)
    invg = (inv + boff).reshape(-1)

    p_pad = jnp.concatenate(
        [p, jnp.zeros((B, N, 125), jnp.float32)], axis=-1).reshape(B * N, 128)
    ps = _k2(p_pad, permg).reshape(B, N, 128)[..., :3]
    xs = ps[..., 0]

    ntiles = N // _K0_Q
    xt = xs.reshape(B, ntiles, _K0_Q)
    margin = jnp.float32(1e-5)
    lo_b = (xt[:, :, 0] - jnp.float32(RADIUS)) - margin
    hi_b = (xt[:, :, -1] + jnp.float32(RADIUS)) + margin
    cnt_lo = jnp.sum(xs[:, None, :] < lo_b[:, :, None], axis=-1)
    cnt_hi = jnp.sum(xs[:, None, :] <= hi_b[:, :, None], axis=-1)
    cb_lo = (cnt_lo // _K0_JB).astype(jnp.int32).reshape(-1)
    cb_hi = ((cnt_hi + _K0_JB - 1) // _K0_JB).astype(jnp.int32).reshape(-1)

    Gfull, Hq_s = _k1(p, ps, fT, w3, wf)

    idx = _k0s(jnp.transpose(ps, (0, 2, 1)),
               perm.astype(jnp.float32).reshape(B, 1, N),
               cb_lo, cb_hi, NSAMPLE)
    idx_flat = idx.reshape(B * N * NSAMPLE)

    gathered = _k2(Gfull.reshape(B * N, C),
                   idx_flat).reshape(B, N, NSAMPLE, C)

    ymax_s, ysum, ysq = _k3(gathered, Hq_s)
    fT_s = _k2(fT.reshape(B * N, C), permg)

    out_s = _k456(ymax_s.reshape(B * N, C), fT_s,
                  ysum.reshape(-1, C), ysq.reshape(-1, C),
                  jnp.transpose(W1), jnp.transpose(W2),
                  g0, b0, g1, b1, g2, b2)
    out_rows = _k2(out_s, invg)
    f_out = jnp.transpose(out_rows.reshape(B, N, C), (0, 2, 1))
    return (p, f_out)


# speedup vs baseline: 1.3083x; 1.3083x over previous
import functools

import jax
import jax.numpy as jnp
from jax import lax
from jax.experimental import pallas as pl
from jax.experimental.pallas import tpu as pltpu
from jax.experimental.pallas import tpu_sc as plsc

RADIUS = 0.1
NSAMPLE = 16
EPS = 1e-5


def _k1_body(p_ref, ps_ref, ft_ref, w3_ref, wf_ref, g_ref, hq_ref):
    pb = p_ref[0]
    psb = ps_ref[0]
    ftb = ft_ref[0]
    gp = jnp.dot(pb, w3_ref[...], preferred_element_type=jnp.float32)
    g_ref[0] = gp + jnp.dot(ftb, wf_ref[...],
                            preferred_element_type=jnp.float32)
    hq_ref[0] = jnp.dot(psb, w3_ref[...], preferred_element_type=jnp.float32)


def _k1(p, ps, fT, w3, wf):
    B, N, C = fT.shape
    Q = 1024
    grid = (B, N // Q)
    return pl.pallas_call(
        _k1_body,
        grid=grid,
        in_specs=[
            pl.BlockSpec((1, Q, 3), lambda b, q: (b, q, 0)),
            pl.BlockSpec((1, Q, 3), lambda b, q: (b, q, 0)),
            pl.BlockSpec((1, Q, C), lambda b, q: (b, q, 0)),
            pl.BlockSpec((3, C), lambda b, q: (0, 0)),
            pl.BlockSpec((C, C), lambda b, q: (0, 0)),
        ],
        out_specs=[
            pl.BlockSpec((1, Q, C), lambda b, q: (b, q, 0)),
            pl.BlockSpec((1, Q, C), lambda b, q: (b, q, 0)),
        ],
        out_shape=[
            jax.ShapeDtypeStruct((B, N, C), jnp.float32),
            jax.ShapeDtypeStruct((B, N, C), jnp.float32),
        ],
    )(p, ps, fT, w3, wf)


def _k2(table, idx_flat):
    info = plsc.get_sparse_core_info()
    nw = info.num_cores * info.num_subcores
    R = idx_flat.shape[0]
    D = table.shape[1]
    per_w = R // nw
    CH = 256
    nch = per_w // CH
    nc = info.num_cores
    mesh = plsc.VectorSubcoreMesh(core_axis_name="c", subcore_axis_name="s")

    @functools.partial(
        pl.kernel, mesh=mesh,
        out_type=jax.ShapeDtypeStruct((R, D), jnp.float32),
        scratch_types=[
            pltpu.VMEM((CH,), jnp.int32),
            pltpu.VMEM((CH, D), jnp.float32),
            pltpu.SemaphoreType.DMA,
        ],
    )
    def k(table_hbm, idx_hbm, out_hbm, idx_v, rows_v, sem):
        wid = lax.axis_index("s") * nc + lax.axis_index("c")
        base0 = wid * per_w

        def body(c, carry):
            base = base0 + c * CH
            pltpu.sync_copy(idx_hbm.at[pl.ds(base, CH)], idx_v)
            pltpu.async_copy(table_hbm.at[idx_v], rows_v, sem).wait()
            pltpu.sync_copy(rows_v, out_hbm.at[pl.ds(base, CH)])
            return carry

        lax.fori_loop(0, nch, body, 0)

    return k(table, idx_flat)


def _k3_body(g_ref, hq_ref, ymax_ref, ysum_ref, ysq_ref):
    g = g_ref[0]
    hq = hq_ref[0]
    gmax = jnp.max(g, axis=1)
    gsum = jnp.sum(g, axis=1)
    gsq = jnp.sum(g * g, axis=1)
    ymax_ref[0] = gmax - hq
    k = jnp.float32(NSAMPLE)
    ysum_ref[0] = jnp.sum(gsum - k * hq, axis=0, keepdims=True)
    ysq_ref[0] = jnp.sum(gsq - 2.0 * hq * gsum + k * hq * hq,
                         axis=0, keepdims=True)


def _k3(gathered, Hq):
    B, N, K, C = gathered.shape
    Q = 512
    nq = N // Q
    grid = (B, nq)
    return pl.pallas_call(
        _k3_body,
        grid=grid,
        in_specs=[
            pl.BlockSpec((1, Q, K, C), lambda b, q: (b, q, 0, 0)),
            pl.BlockSpec((1, Q, C), lambda b, q: (b, q, 0)),
        ],
        out_specs=[
            pl.BlockSpec((1, Q, C), lambda b, q: (b, q, 0)),
            pl.BlockSpec((1, 1, C), lambda b, q, _nq=nq: (b * _nq + q, 0, 0)),
            pl.BlockSpec((1, 1, C), lambda b, q, _nq=nq: (b * _nq + q, 0, 0)),
        ],
        out_shape=[
            jax.ShapeDtypeStruct((B, N, C), jnp.float32),
            jax.ShapeDtypeStruct((B * nq, 1, C), jnp.float32),
            jax.ShapeDtypeStruct((B * nq, 1, C), jnp.float32),
        ],
    )(gathered, Hq)


def _k456_body(ymax_ref, ft_ref, ysum_ref, ysq_ref, w1t_ref, w2t_ref,
               g0_ref, b0_ref, g1_ref, b1_ref, g2_ref, b2_ref, out_ref,
               *, n0, n1):
    ymax = ymax_ref[...]
    ft = ft_ref[...]
    m0 = jnp.sum(ysum_ref[...], axis=0, keepdims=True) / n0
    v0 = jnp.sum(ysq_ref[...], axis=0, keepdims=True) / n0 - m0 * m0
    r0 = g0_ref[...] * lax.rsqrt(v0 + EPS)
    fa = jnp.maximum((ymax - m0) * r0 + b0_ref[...], 0.0)
    h1 = jnp.dot(fa, w1t_ref[...], preferred_element_type=jnp.float32)
    m1 = jnp.sum(h1, axis=0, keepdims=True) / n1
    v1 = jnp.sum(h1 * h1, axis=0, keepdims=True) / n1 - m1 * m1
    r1 = g1_ref[...] * lax.rsqrt(v1 + EPS)
    h1n = jnp.maximum((h1 - m1) * r1 + b1_ref[...], 0.0)
    h2 = jnp.dot(h1n, w2t_ref[...], preferred_element_type=jnp.float32)
    m2 = jnp.sum(h2, axis=0, keepdims=True) / n1
    v2 = jnp.sum(h2 * h2, axis=0, keepdims=True) / n1 - m2 * m2
    r2 = g2_ref[...] * lax.rsqrt(v2 + EPS)
    h2n = (h2 - m2) * r2 + b2_ref[...]
    out_ref[...] = jnp.maximum(h2n + ft, 0.0)


def _k456(ymax, ftf, ysum, ysq, w1t, w2t, g0, b0, g1, b1, g2, b2):
    M, C = ymax.shape
    n0 = float(M * NSAMPLE)
    n1 = float(M)
    row = lambda v: v.reshape(1, C)
    return pl.pallas_call(
        functools.partial(_k456_body, n0=n0, n1=n1),
        out_shape=jax.ShapeDtypeStruct((M, C), jnp.float32),
    )(ymax, ftf, ysum, ysq, w1t, w2t,
      row(g0), row(b0), row(g1), row(b1), row(g2), row(b2))


def _merge16(pos, blk):
    m = [jnp.minimum(pos[i], blk[15 - i]) for i in range(16)]
    for stride in (8, 4, 2, 1):
        nm = list(m)
        for i in range(16):
            if (i % (2 * stride)) < stride:
                lo = jnp.minimum(m[i], m[i + stride])
                hi = jnp.maximum(m[i], m[i + stride])
                nm[i] = lo
                nm[i + stride] = hi
        m = nm
    return m


def _k0s_body(lo_ref, hi_ref, pts_ref, ido_ref, idx_ref,
              *, Q, JB, N, W, nsample):
    b = pl.program_id(0)
    qt = pl.program_id(1)
    q = pts_ref[0, :, pl.ds(qt * Q, Q)]
    qx = q[0, :].reshape(Q, 1)
    qy = q[1, :].reshape(Q, 1)
    qz = q[2, :].reshape(Q, 1)
    r2 = jnp.float32(RADIUS * RADIUS)
    big = jnp.float32(N)

    def scan_block(start, pos):
        cols = pts_ref[0, :, pl.ds(start, JB)]
        oid = ido_ref[0, 0:1, pl.ds(start, JB)]
        cx = cols[0, :].reshape(1, JB)
        cy = cols[1, :].reshape(1, JB)
        cz = cols[2, :].reshape(1, JB)
        dx = qx - cx
        dy = qy - cy
        dz = qz - cz
        d2 = dx * dx + dy * dy + dz * dz
        ov = jnp.where(d2 <= r2, oid, big)
        prev = jnp.full((Q, 1), jnp.float32(-1.0))
        blk = []
        for _ in range(nsample):
            hit = jnp.where(ov > prev, ov, big)
            cand = jnp.min(hit, axis=1, keepdims=True)
            blk.append(cand)
            prev = cand
        return tuple(_merge16(list(pos), blk))

    init = tuple(jnp.full((Q, 1), big) for _ in range(nsample))
    t = b * (pl.num_programs(1)) + qt
    pos = lax.fori_loop(lo_ref[t], hi_ref[t],
                        lambda cb, pp: scan_block(cb * JB, pp), init)
    first = pos[0]
    base = b * N
    for s in range(nsample):
        sel = jnp.where(pos[s] < big, pos[s], first)
        idx_ref[0, :, s:s + 1] = sel.astype(jnp.int32) + base


def _k0s(pTs, ido, cb_lo, cb_hi, nsample):
    B, _, N = pTs.shape
    Q = _K0_Q
    JB = _K0_JB
    grid = (B, N // Q)
    return pl.pallas_call(
        functools.partial(_k0s_body, Q=Q, JB=JB, N=N, W=_K0_W,
                          nsample=nsample),
        grid=grid,
        in_specs=[
            pl.BlockSpec(memory_space=pltpu.SMEM),
            pl.BlockSpec(memory_space=pltpu.SMEM),
            pl.BlockSpec((1, 3, N), lambda b, q: (b, 0, 0)),
            pl.BlockSpec((1, 1, N), lambda b, q: (b, 0, 0)),
        ],
        out_specs=pl.BlockSpec((1, Q, nsample), lambda b, q: (b, q, 0)),
        out_shape=jax.ShapeDtypeStruct((B, N, nsample), jnp.int32),
    )(cb_lo, cb_hi, pTs, ido)


def _ball_query_xla(p, radius, nsample):
    B, N, _ = p.shape
    r2 = radius * radius
    order = jnp.arange(N, dtype=jnp.float32)[None, None, :]
    chunk = 1024
    idx_chunks = []
    for s in range(0, N, chunk):
        q = p[:, s:s + chunk]
        d2 = jnp.sum((q[:, :, None, :] - p[:, None, :, :]) ** 2, axis=-1)
        valid = d2 <= r2
        score = jnp.where(valid, order, float(N))
        neg_vals, idx = lax.top_k(-score, nsample)
        found = (-neg_vals) < float(N)
        firstc = idx[:, :, :1]
        idx = jnp.where(found, idx, firstc)
        idx_chunks.append(idx)
    return jnp.concatenate(idx_chunks, axis=1)


_K0_Q = 1024
_K0_JB = 1024
_K0_W = 4096


def kernel(p, f, W0, g0, b0, W1, g1, b1, W2, g2, b2):
    B, C, N = f.shape
    fT = jnp.transpose(f, (0, 2, 1))
    w3 = jnp.transpose(W0[:, :3])
    wf = jnp.transpose(W0[:, 3:])

    boff = (jnp.arange(B, dtype=jnp.int32) * N)[:, None]
    perm = jnp.argsort(p[..., 0], axis=1).astype(jnp.int32)
    permg = (perm + boff).reshape(-1)
    inv = jnp.argsort(perm, axis=1).astype(jnp.int32)
    invg = (inv + boff).reshape(-1)

    p_pad = jnp.concatenate(
        [p, jnp.zeros((B, N, 125), jnp.float32)], axis=-1).reshape(B * N, 128)
    ps = _k2(p_pad, permg).reshape(B, N, 128)[..., :3]
    xs = ps[..., 0]

    ntiles = N // _K0_Q
    xt = xs.reshape(B, ntiles, _K0_Q)
    margin = jnp.float32(1e-5)
    lo_b = (xt[:, :, 0] - jnp.float32(RADIUS)) - margin
    hi_b = (xt[:, :, -1] + jnp.float32(RADIUS)) + margin
    cnt_lo = jnp.sum(xs[:, None, :] < lo_b[:, :, None], axis=-1)
    cnt_hi = jnp.sum(xs[:, None, :] <= hi_b[:, :, None], axis=-1)
    cb_lo = (cnt_lo // _K0_JB).astype(jnp.int32).reshape(-1)
    cb_hi = ((cnt_hi + _K0_JB - 1) // _K0_JB).astype(jnp.int32).reshape(-1)

    Gfull, Hq_s = _k1(p, ps, fT, w3, wf)

    idx = _k0s(jnp.transpose(ps, (0, 2, 1)),
               perm.astype(jnp.float32).reshape(B, 1, N),
               cb_lo, cb_hi, NSAMPLE)
    idx_flat = idx.reshape(B * N * NSAMPLE)

    gathered = _k2(Gfull.reshape(B * N, C),
                   idx_flat).reshape(B, N, NSAMPLE, C)

    ymax_s, ysum, ysq = _k3(gathered, Hq_s)
    fT_s = _k2(fT.reshape(B * N, C), permg)

    out_s = _k456(ymax_s.reshape(B * N, C), fT_s,
                  ysum.reshape(-1, C), ysq.reshape(-1, C),
                  jnp.transpose(W1), jnp.transpose(W2),
                  g0, b0, g1, b1, g2, b2)
    out_rows = _k2(out_s, invg)
    f_out = jnp.transpose(out_rows.reshape(B, N, C), (0, 2, 1))
    return (p, f_out)


# speedup vs baseline: 1.3337x; 1.0194x over previous
import functools

import jax
import jax.numpy as jnp
from jax import lax
from jax.experimental import pallas as pl
from jax.experimental.pallas import tpu as pltpu
from jax.experimental.pallas import tpu_sc as plsc

RADIUS = 0.1
NSAMPLE = 16
EPS = 1e-5


def _k1_body(p_ref, ps_ref, ft_ref, w3_ref, wf_ref, g_ref, hq_ref):
    pb = p_ref[0]
    psb = ps_ref[0]
    ftb = ft_ref[0]
    gp = jnp.dot(pb, w3_ref[...], preferred_element_type=jnp.float32)
    g_ref[0] = gp + jnp.dot(ftb, wf_ref[...],
                            preferred_element_type=jnp.float32)
    hq_ref[0] = jnp.dot(psb, w3_ref[...], preferred_element_type=jnp.float32)


def _k1(p, ps, fT, w3, wf):
    B, N, C = fT.shape
    Q = 1024
    grid = (B, N // Q)
    return pl.pallas_call(
        _k1_body,
        grid=grid,
        in_specs=[
            pl.BlockSpec((1, Q, 3), lambda b, q: (b, q, 0)),
            pl.BlockSpec((1, Q, 3), lambda b, q: (b, q, 0)),
            pl.BlockSpec((1, Q, C), lambda b, q: (b, q, 0)),
            pl.BlockSpec((3, C), lambda b, q: (0, 0)),
            pl.BlockSpec((C, C), lambda b, q: (0, 0)),
        ],
        out_specs=[
            pl.BlockSpec((1, Q, C), lambda b, q: (b, q, 0)),
            pl.BlockSpec((1, Q, C), lambda b, q: (b, q, 0)),
        ],
        out_shape=[
            jax.ShapeDtypeStruct((B, N, C), jnp.float32),
            jax.ShapeDtypeStruct((B, N, C), jnp.float32),
        ],
    )(p, ps, fT, w3, wf)


def _k2(table, idx_flat):
    info = plsc.get_sparse_core_info()
    nw = info.num_cores * info.num_subcores
    R = idx_flat.shape[0]
    D = table.shape[1]
    per_w = R // nw
    CH = 256
    nch = per_w // CH
    nc = info.num_cores
    mesh = plsc.VectorSubcoreMesh(core_axis_name="c", subcore_axis_name="s")

    @functools.partial(
        pl.kernel, mesh=mesh,
        out_type=jax.ShapeDtypeStruct((R, D), jnp.float32),
        scratch_types=[
            pltpu.VMEM((CH,), jnp.int32),
            pltpu.VMEM((CH, D), jnp.float32),
            pltpu.SemaphoreType.DMA,
        ],
    )
    def k(table_hbm, idx_hbm, out_hbm, idx_v, rows_v, sem):
        wid = lax.axis_index("s") * nc + lax.axis_index("c")
        base0 = wid * per_w

        def body(c, carry):
            base = base0 + c * CH
            pltpu.sync_copy(idx_hbm.at[pl.ds(base, CH)], idx_v)
            pltpu.async_copy(table_hbm.at[idx_v], rows_v, sem).wait()
            pltpu.sync_copy(rows_v, out_hbm.at[pl.ds(base, CH)])
            return carry

        lax.fori_loop(0, nch, body, 0)

    return k(table, idx_flat)


def _k3_body(g_ref, hq_ref, ymax_ref, ysum_ref, ysq_ref):
    g = g_ref[0]
    hq = hq_ref[0]
    gmax = jnp.max(g, axis=1)
    gsum = jnp.sum(g, axis=1)
    gsq = jnp.sum(g * g, axis=1)
    ymax_ref[0] = gmax - hq
    k = jnp.float32(NSAMPLE)
    ysum_ref[0] = jnp.sum(gsum - k * hq, axis=0, keepdims=True)
    ysq_ref[0] = jnp.sum(gsq - 2.0 * hq * gsum + k * hq * hq,
                         axis=0, keepdims=True)


def _k3(gathered, Hq):
    B, N, K, C = gathered.shape
    Q = 512
    nq = N // Q
    grid = (B, nq)
    return pl.pallas_call(
        _k3_body,
        grid=grid,
        in_specs=[
            pl.BlockSpec((1, Q, K, C), lambda b, q: (b, q, 0, 0)),
            pl.BlockSpec((1, Q, C), lambda b, q: (b, q, 0)),
        ],
        out_specs=[
            pl.BlockSpec((1, Q, C), lambda b, q: (b, q, 0)),
            pl.BlockSpec((1, 1, C), lambda b, q, _nq=nq: (b * _nq + q, 0, 0)),
            pl.BlockSpec((1, 1, C), lambda b, q, _nq=nq: (b * _nq + q, 0, 0)),
        ],
        out_shape=[
            jax.ShapeDtypeStruct((B, N, C), jnp.float32),
            jax.ShapeDtypeStruct((B * nq, 1, C), jnp.float32),
            jax.ShapeDtypeStruct((B * nq, 1, C), jnp.float32),
        ],
    )(gathered, Hq)


def _k456_body(ymax_ref, ft_ref, ysum_ref, ysq_ref, w1t_ref, w2t_ref,
               g0_ref, b0_ref, g1_ref, b1_ref, g2_ref, b2_ref, out_ref,
               *, n0, n1):
    ymax = ymax_ref[...]
    ft = ft_ref[...]
    m0 = jnp.sum(ysum_ref[...], axis=0, keepdims=True) / n0
    v0 = jnp.sum(ysq_ref[...], axis=0, keepdims=True) / n0 - m0 * m0
    r0 = g0_ref[...] * lax.rsqrt(v0 + EPS)
    fa = jnp.maximum((ymax - m0) * r0 + b0_ref[...], 0.0)
    h1 = jnp.dot(fa, w1t_ref[...], preferred_element_type=jnp.float32)
    m1 = jnp.sum(h1, axis=0, keepdims=True) / n1
    v1 = jnp.sum(h1 * h1, axis=0, keepdims=True) / n1 - m1 * m1
    r1 = g1_ref[...] * lax.rsqrt(v1 + EPS)
    h1n = jnp.maximum((h1 - m1) * r1 + b1_ref[...], 0.0)
    h2 = jnp.dot(h1n, w2t_ref[...], preferred_element_type=jnp.float32)
    m2 = jnp.sum(h2, axis=0, keepdims=True) / n1
    v2 = jnp.sum(h2 * h2, axis=0, keepdims=True) / n1 - m2 * m2
    r2 = g2_ref[...] * lax.rsqrt(v2 + EPS)
    h2n = (h2 - m2) * r2 + b2_ref[...]
    out_ref[...] = jnp.maximum(h2n + ft, 0.0)


def _k456(ymax, ftf, ysum, ysq, w1t, w2t, g0, b0, g1, b1, g2, b2):
    M, C = ymax.shape
    n0 = float(M * NSAMPLE)
    n1 = float(M)
    row = lambda v: v.reshape(1, C)
    return pl.pallas_call(
        functools.partial(_k456_body, n0=n0, n1=n1),
        out_shape=jax.ShapeDtypeStruct((M, C), jnp.float32),
    )(ymax, ftf, ysum, ysq, w1t, w2t,
      row(g0), row(b0), row(g1), row(b1), row(g2), row(b2))


def _merge16(pos, blk):
    m = [jnp.minimum(pos[i], blk[15 - i]) for i in range(16)]
    for stride in (8, 4, 2, 1):
        nm = list(m)
        for i in range(16):
            if (i % (2 * stride)) < stride:
                lo = jnp.minimum(m[i], m[i + stride])
                hi = jnp.maximum(m[i], m[i + stride])
                nm[i] = lo
                nm[i + stride] = hi
        m = nm
    return m


def _k0s_body(lo_ref, hi_ref, pts_ref, ido_ref, idx_ref,
              *, Q, JB, N, nsample):
    b = pl.program_id(0)
    qt = pl.program_id(1)
    q = pts_ref[0, :, pl.ds(qt * Q, Q)]
    qx = q[0, :].reshape(Q, 1)
    qy = q[1, :].reshape(Q, 1)
    qz = q[2, :].reshape(Q, 1)
    r2 = jnp.float32(RADIUS * RADIUS)
    big = jnp.float32(N)

    def scan_block(start, pos):
        cols = pts_ref[0, :, pl.ds(start, JB)]
        oid = ido_ref[0, 0:1, pl.ds(start, JB)]
        cx = cols[0, :].reshape(1, JB)
        cy = cols[1, :].reshape(1, JB)
        cz = cols[2, :].reshape(1, JB)
        dx = qx - cx
        dy = qy - cy
        dz = qz - cz
        d2 = dx * dx + dy * dy + dz * dz
        ov = jnp.where(d2 <= r2, oid, big)
        prev = jnp.full((Q, 1), jnp.float32(-1.0))
        blk = []
        for _ in range(nsample):
            hit = jnp.where(ov > prev, ov, big)
            cand = jnp.min(hit, axis=1, keepdims=True)
            blk.append(cand)
            prev = cand
        return tuple(_merge16(list(pos), blk))

    init = tuple(jnp.full((Q, 1), big) for _ in range(nsample))
    t = b * (pl.num_programs(1)) + qt
    pos = lax.fori_loop(lo_ref[t], hi_ref[t],
                        lambda cb, pp: scan_block(cb * JB, pp), init)
    first = pos[0]
    base = b * N
    for s in range(nsample):
        sel = jnp.where(pos[s] < big, pos[s], first)
        idx_ref[0, :, s:s + 1] = sel.astype(jnp.int32) + base


def _k0s(pTs, ido, cb_lo, cb_hi, nsample):
    B, _, N = pTs.shape
    Q = _K0_Q
    JB = _K0_JB
    grid = (B, N // Q)
    return pl.pallas_call(
        functools.partial(_k0s_body, Q=Q, JB=JB, N=N, nsample=nsample),
        grid=grid,
        in_specs=[
            pl.BlockSpec(memory_space=pltpu.SMEM),
            pl.BlockSpec(memory_space=pltpu.SMEM),
            pl.BlockSpec((1, 3, N), lambda b, q: (b, 0, 0)),
            pl.BlockSpec((1, 1, N), lambda b, q: (b, 0, 0)),
        ],
        out_specs=pl.BlockSpec((1, Q, nsample), lambda b, q: (b, q, 0)),
        out_shape=jax.ShapeDtypeStruct((B, N, nsample), jnp.int32),
    )(cb_lo, cb_hi, pTs, ido)


_K0_Q = 1024
_K0_JB = 1024


def kernel(p, f, W0, g0, b0, W1, g1, b1, W2, g2, b2):
    B, C, N = f.shape
    fT = jnp.transpose(f, (0, 2, 1))
    w3 = jnp.transpose(W0[:, :3])
    wf = jnp.transpose(W0[:, 3:])

    boff = (jnp.arange(B, dtype=jnp.int32) * N)[:, None]
    perm = jnp.argsort(p[..., 0], axis=1).astype(jnp.int32)
    permg = (perm + boff).reshape(-1)
    row_ids = jnp.arange(N, dtype=jnp.int32)
    inv = jax.vmap(
        lambda pm: jnp.zeros((N,), jnp.int32).at[pm].set(row_ids))(perm)
    invg = (inv + boff).reshape(-1)

    p_pad = jnp.concatenate(
        [p, jnp.zeros((B, N, 125), jnp.float32)], axis=-1).reshape(B * N, 128)
    ps = _k2(p_pad, permg).reshape(B, N, 128)[..., :3]
    xs = ps[..., 0]

    ntiles = N // _K0_Q
    xt = xs.reshape(B, ntiles, _K0_Q)
    margin = jnp.float32(1e-5)
    lo_b = (xt[:, :, 0] - jnp.float32(RADIUS)) - margin
    hi_b = (xt[:, :, -1] + jnp.float32(RADIUS)) + margin
    cnt_lo = jnp.sum(xs[:, None, :] < lo_b[:, :, None], axis=-1)
    cnt_hi = jnp.sum(xs[:, None, :] <= hi_b[:, :, None], axis=-1)
    cb_lo = (cnt_lo // _K0_JB).astype(jnp.int32).reshape(-1)
    cb_hi = ((cnt_hi + _K0_JB - 1) // _K0_JB).astype(jnp.int32).reshape(-1)

    Gfull, Hq_s = _k1(p, ps, fT, w3, wf)

    idx = _k0s(jnp.transpose(ps, (0, 2, 1)),
               perm.astype(jnp.float32).reshape(B, 1, N),
               cb_lo, cb_hi, NSAMPLE)
    idx_flat = idx.reshape(B * N * NSAMPLE)

    gathered = _k2(Gfull.reshape(B * N, C),
                   idx_flat).reshape(B, N, NSAMPLE, C)

    ymax_s, ysum, ysq = _k3(gathered, Hq_s)
    fT_s = _k2(fT.reshape(B * N, C), permg)

    out_s = _k456(ymax_s.reshape(B * N, C), fT_s,
                  ysum.reshape(-1, C), ysq.reshape(-1, C),
                  jnp.transpose(W1), jnp.transpose(W2),
                  g0, b0, g1, b1, g2, b2)
    out_rows = _k2(out_s, invg)
    f_out = jnp.transpose(out_rows.reshape(B, N, C), (0, 2, 1))
    return (p, f_out)
